# baseline (speedup 1.0000x reference)
import functools

import jax
import jax.numpy as jnp
from jax import lax
from jax.experimental import pallas as pl
from jax.experimental.pallas import tpu as pltpu
from jax.experimental.pallas import tpu_sc as plsc

_F = 26
_K = 16
_CARD = 100000


def _build_fm_call(B):
    info = plsc.get_sparse_core_info()
    NC, NS = info.num_cores, info.num_subcores
    NW = NC * NS
    bw = B // NW
    CB = 64
    NCH = bw // CB
    CI = CB * _F
    GW = 128
    NG = CI // GW
    assert B % NW == 0 and bw % CB == 0 and CI % GW == 0 and NCH % 2 == 0

    mesh = plsc.VectorSubcoreMesh(core_axis_name="c", subcore_axis_name="s")

    @functools.partial(
        pl.kernel,
        mesh=mesh,
        compiler_params=pltpu.CompilerParams(use_tc_tiling_on_sc=False),
        out_type=jax.ShapeDtypeStruct((B,), jnp.float32),
        scratch_types=[
            pltpu.VMEM((CI,), jnp.int32),
            pltpu.VMEM((CI,), jnp.int32),
            pltpu.VMEM((CI, _K), jnp.float32),
            pltpu.VMEM((CI, _K), jnp.float32),
            pltpu.VMEM((CI + 16,), jnp.float32),
            pltpu.VMEM((CI + 16,), jnp.float32),
            pltpu.VMEM((CI,), jnp.int32),
            pltpu.VMEM((CB,), jnp.float32),
            pltpu.SemaphoreType.DMA,
            pltpu.SemaphoreType.DMA,
        ],
    )
    def fm(x_hbm, offs_hbm, emb_hbm, lin_hbm, out_hbm,
           idx0, idx1, rows0, rows1, linv0, linv1, offs_v, out_v,
           sem0, sem1):
        wid = lax.axis_index("s") * NC + lax.axis_index("c")
        base = wid * bw

        pltpu.sync_copy(offs_hbm, offs_v)
        zero16 = jnp.zeros((16,), jnp.float32)
        linv0[pl.ds(CI, 16)] = zero16
        linv1[pl.ds(CI, 16)] = zero16
        lane = lax.iota(jnp.int32, 16)
        tail_mask = jnp.where(lane < (_F - 16), 1.0, 0.0)

        dnums = lax.GatherDimensionNumbers(
            offset_dims=(), collapsed_slice_dims=(0,), start_index_map=(0,))
        rot_idx = [((lane + sh) & 15)[:, None] for sh in (8, 4, 2, 1)]

        def lane_sum(v):
            for ri in rot_idx:
                v = v + lax.gather(
                    v, ri, dnums, (1,),
                    mode=lax.GatherScatterMode.PROMISE_IN_BOUNDS)
            return v

        idx = (idx0, idx1)
        rows = (rows0, rows1)
        linv = (linv0, linv1)
        sems = (sem0, sem1)

        def stage(c, s):
            start = (base + c * CB) * _F
            pltpu.sync_copy(x_hbm.at[pl.ds(start, CI)], idx[s])

            def add_offs(i, carry):
                sl = pl.ds(pl.multiple_of(i * 16, 16), 16)
                idx[s][sl] = idx[s][sl] + offs_v[sl]
                return carry

            lax.fori_loop(0, CI // 16, add_offs, 0)
            for g in range(NG):
                gs = pl.ds(g * GW, GW)
                pltpu.make_async_copy(
                    emb_hbm.at[idx[s].at[gs]], rows[s].at[gs, :], sems[s]
                ).start()
                pltpu.make_async_copy(
                    lin_hbm.at[idx[s].at[gs]], linv[s].at[gs], sems[s]
                ).start()

        def drain(s):
            for g in range(NG):
                gs = pl.ds(g * GW, GW)
                pltpu.make_async_copy(
                    emb_hbm.at[idx[s].at[gs]], rows[s].at[gs, :], sems[s]
                ).wait()
                pltpu.make_async_copy(
                    lin_hbm.at[idx[s].at[gs]], linv[s].at[gs], sems[s]
                ).wait()

        def compute(c, s):
            r_ref = rows[s]
            l_ref = linv[s]

            def grp(g, carry):
                res = zero16
                for j in range(16):
                    roff = (g * 16 + j) * _F
                    r = r_ref[roff]
                    acc_s = r
                    acc_q = r * r
                    for f in range(1, _F):
                        r = r_ref[roff + f]
                        acc_s = acc_s + r
                        acc_q = acc_q + r * r
                    t = acc_s * acc_s - acc_q
                    v0 = l_ref[pl.ds(roff, 16)]
                    v1 = l_ref[pl.ds(roff + 16, 16)]
                    val = lane_sum(0.5 * t + v0 + tail_mask * v1)
                    res = jnp.where(lane == j, val, res)
                sl = pl.ds(pl.multiple_of(g * 16, 16), 16)
                out_v[sl] = res
                return carry

            lax.fori_loop(0, CB // 16, grp, 0)
            pltpu.sync_copy(out_v, out_hbm.at[pl.ds(base + c * CB, CB)])

        stage(0, 0)

        def pipe(i, carry):
            c = i * 2
            stage(c + 1, 1)
            drain(0)
            compute(c, 0)

            @pl.when(c + 2 < NCH)
            def _():
                stage(c + 2, 0)

            drain(1)
            compute(c + 1, 1)
            return carry

        lax.fori_loop(0, NCH // 2, pipe, 0)

    return fm


def _tc_transpose(emb_t):
    K, R = emb_t.shape
    BLK = 4096

    def body(in_ref, out_ref):
        out_ref[...] = in_ref[...].T

    return pl.pallas_call(
        body,
        grid=(pl.cdiv(R, BLK),),
        in_specs=[pl.BlockSpec((K, BLK), lambda g: (0, g))],
        out_specs=pl.BlockSpec((BLK, K), lambda g: (g, 0)),
        out_shape=jax.ShapeDtypeStruct((R, K), jnp.float32),
    )(emb_t)


def kernel(x, emb_table, linear_table, bias):
    B, F = x.shape
    emb_rm = _tc_transpose(jnp.swapaxes(emb_table, 0, 1))
    x_flat = x.reshape(-1).astype(jnp.int32)
    offsets = jnp.arange(F, dtype=jnp.int32) * _CARD
    offs_pat = jnp.tile(offsets, 64)
    out = _build_fm_call(B)(x_flat, offs_pat, emb_rm, linear_table)
    return out.reshape(B, 1) + bias[None, :]


# speedup vs baseline: 4.5590x; 4.5590x over previous
import functools

import jax
import jax.numpy as jnp
from jax import lax
from jax.experimental import pallas as pl
from jax.experimental.pallas import tpu as pltpu
from jax.experimental.pallas import tpu_sc as plsc

_F = 26
_K = 16
_CARD = 100000
_CB = 128


def _build_fm_call(B):
    info = plsc.get_sparse_core_info()
    NC, NS = info.num_cores, info.num_subcores
    NW = NC * NS
    bw = B // NW
    CB = _CB
    NCH = bw // CB
    CI = CB * _F
    GW = 128
    NG = CI // GW
    assert B % NW == 0 and bw % CB == 0 and CI % GW == 0 and NCH % 2 == 0

    mesh = plsc.VectorSubcoreMesh(core_axis_name="c", subcore_axis_name="s")

    @functools.partial(
        pl.kernel,
        mesh=mesh,
        compiler_params=pltpu.CompilerParams(use_tc_tiling_on_sc=False),
        out_type=jax.ShapeDtypeStruct((B,), jnp.float32),
        scratch_types=[
            pltpu.VMEM((CI,), jnp.int32),
            pltpu.VMEM((CI,), jnp.int32),
            pltpu.VMEM((CI, _K), jnp.float32),
            pltpu.VMEM((CI, _K), jnp.float32),
            pltpu.VMEM((CI + 16,), jnp.float32),
            pltpu.VMEM((CI + 16,), jnp.float32),
            pltpu.VMEM((CI,), jnp.int32),
            pltpu.VMEM((CB,), jnp.float32),
            pltpu.SemaphoreType.DMA,
            pltpu.SemaphoreType.DMA,
        ],
    )
    def fm(x_hbm, offs_hbm, emb_hbm, lin_hbm, out_hbm,
           idx0, idx1, rows0, rows1, linv0, linv1, offs_v, out_v,
           sem0, sem1):
        wid = lax.axis_index("s") * NC + lax.axis_index("c")
        base = wid * bw

        pltpu.sync_copy(offs_hbm, offs_v)
        zero16 = jnp.zeros((16,), jnp.float32)
        linv0[pl.ds(CI, 16)] = zero16
        linv1[pl.ds(CI, 16)] = zero16
        lane = lax.iota(jnp.int32, 16)
        tail_mask = jnp.where(lane < (_F - 16), 1.0, 0.0)

        dnums = lax.GatherDimensionNumbers(
            offset_dims=(), collapsed_slice_dims=(0,), start_index_map=(0,))
        rot_idx = [((lane + sh) & 15)[:, None] for sh in (8, 4, 2, 1)]

        def lane_sum(v):
            for ri in rot_idx:
                v = v + lax.gather(
                    v, ri, dnums, (1,),
                    mode=lax.GatherScatterMode.PROMISE_IN_BOUNDS)
            return v

        idx = (idx0, idx1)
        rows = (rows0, rows1)
        linv = (linv0, linv1)
        sems = (sem0, sem1)

        def stage(c, s):
            start = (base + c * CB) * _F
            pltpu.sync_copy(x_hbm.at[pl.ds(start, CI)], idx[s])

            def add_offs(i, carry):
                sl = pl.ds(pl.multiple_of(i * 16, 16), 16)
                idx[s][sl] = idx[s][sl] + offs_v[sl]
                return carry

            lax.fori_loop(0, CI // 16, add_offs, 0)
            for g in range(NG):
                gs = pl.ds(g * GW, GW)
                pltpu.make_async_copy(
                    emb_hbm.at[idx[s].at[gs]], rows[s].at[gs, :], sems[s]
                ).start()
                pltpu.make_async_copy(
                    lin_hbm.at[idx[s].at[gs]], linv[s].at[gs], sems[s]
                ).start()

        def drain(s):
            for g in range(NG):
                gs = pl.ds(g * GW, GW)
                pltpu.make_async_copy(
                    emb_hbm.at[idx[s].at[gs]], rows[s].at[gs, :], sems[s]
                ).wait()
                pltpu.make_async_copy(
                    lin_hbm.at[idx[s].at[gs]], linv[s].at[gs], sems[s]
                ).wait()

        def compute(c, s):
            r_ref = rows[s]
            l_ref = linv[s]

            def grp(g, carry):
                res = zero16
                for j in range(16):
                    roff = (g * 16 + j) * _F
                    r = r_ref[roff]
                    acc_s = r
                    acc_q = r * r
                    for f in range(1, _F):
                        r = r_ref[roff + f]
                        acc_s = acc_s + r
                        acc_q = acc_q + r * r
                    t = acc_s * acc_s - acc_q
                    v0 = l_ref[pl.ds(roff, 16)]
                    v1 = l_ref[pl.ds(roff + 16, 16)]
                    val = lane_sum(0.5 * t + v0 + tail_mask * v1)
                    res = jnp.where(lane == j, val, res)
                sl = pl.ds(pl.multiple_of(g * 16, 16), 16)
                out_v[sl] = res
                return carry

            lax.fori_loop(0, CB // 16, grp, 0)
            pltpu.sync_copy(out_v, out_hbm.at[pl.ds(base + c * CB, CB)])

        stage(0, 0)

        def pipe(i, carry):
            c = i * 2
            stage(c + 1, 1)
            drain(0)
            compute(c, 0)

            @pl.when(c + 2 < NCH)
            def _():
                stage(c + 2, 0)

            drain(1)
            compute(c + 1, 1)
            return carry

        lax.fori_loop(0, NCH // 2, pipe, 0)

    return fm


def _build_untile_call(R):
    info = plsc.get_sparse_core_info()
    NC, NS = info.num_cores, info.num_subcores
    NW = NC * NS
    CW = 1024
    NFULL = R // CW
    REM = R - NFULL * CW
    ITER = -(-NFULL // NW)
    if ITER % 2:
        ITER += 1
    OUTR = R * 16 // 128

    mesh = plsc.VectorSubcoreMesh(core_axis_name="c", subcore_axis_name="s")

    @functools.partial(
        pl.kernel,
        mesh=mesh,
        compiler_params=pltpu.CompilerParams(needs_layout_passes=False),
        out_type=jax.ShapeDtypeStruct((R * 16,), jnp.float32),
        scratch_types=[
            pltpu.VMEM((16, CW), jnp.float32),
            pltpu.VMEM((16, CW), jnp.float32),
            pltpu.VMEM((CW * 16,), jnp.float32),
            pltpu.VMEM((CW * 16,), jnp.float32),
            pltpu.SemaphoreType.DMA,
            pltpu.SemaphoreType.DMA,
            pltpu.SemaphoreType.DMA,
            pltpu.SemaphoreType.DMA,
        ],
    )
    def untile(embt_hbm, tail_hbm, out_hbm, st0, st1, ov0, ov1,
               sem0, sem1, osem0, osem1):
        wid = lax.axis_index("s") * NC + lax.axis_index("c")
        lane = lax.iota(jnp.int32, 16)
        base16 = [lane * 16 + k for k in range(16)]
        sts = (st0, st1)
        ovs = (ov0, ov1)
        sems = (sem0, sem1)
        osems = (osem0, osem1)

        def out_copy(c, s):
            return pltpu.make_async_copy(
                ovs[s], out_hbm.at[pl.ds(c * (CW * 16), CW * 16)], osems[s])

        def start_stage(i, s):
            c = wid + i * NW

            @pl.when(c < NFULL)
            def _():
                pltpu.make_async_copy(
                    embt_hbm.at[:, pl.ds(c * CW, CW)], sts[s], sems[s]
                ).start()

        def extract(s):
            st = sts[s]
            ov = ovs[s]

            def grp(g, carry):
                g256 = g * 256
                sl = pl.ds(pl.multiple_of(g * 16, 16), 16)
                for k in range(16):
                    plsc.store_scatter(ov, [base16[k] + g256], st[k, sl])
                return carry

            lax.fori_loop(0, CW // 16, grp, 0, unroll=2)

        def do_chunk(i, s):
            c = wid + i * NW

            @pl.when(c < NFULL)
            def _():
                pltpu.make_async_copy(
                    embt_hbm.at[:, pl.ds(c * CW, CW)], sts[s], sems[s]
                ).wait()
                cp = c - 2 * NW

                @pl.when(cp >= 0)
                def _():
                    out_copy(cp, s).wait()

                extract(s)
                out_copy(c, s).start()

        start_stage(0, 0)

        def pipe(ih, carry):
            i0 = ih * 2
            start_stage(i0 + 1, 1)
            do_chunk(i0, 0)
            start_stage(i0 + 2, 0)
            do_chunk(i0 + 1, 1)
            return carry

        lax.fori_loop(0, ITER // 2, pipe, 0)

        for i in (ITER - 2, ITER - 1):
            ci = wid + i * NW

            @pl.when(ci < NFULL)
            def _(ci=ci, s=i % 2):
                out_copy(ci, s).wait()

        if REM:
            @pl.when(wid == 0)
            def _():
                pltpu.sync_copy(tail_hbm, ov0.at[pl.ds(0, REM * 16)])
                pltpu.sync_copy(
                    ov0.at[pl.ds(0, REM * 16)],
                    out_hbm.at[pl.ds(NFULL * CW * 16, REM * 16)])

    return untile


def kernel(x, emb_table, linear_table, bias):
    B, F = x.shape
    R = emb_table.shape[0]
    rem = R % 1024
    tail = emb_table[R - rem:, :].reshape(-1)
    stage = _build_untile_call(R)(jnp.swapaxes(emb_table, 0, 1), tail)
    emb_rm = stage.reshape(R, _K)
    x_flat = x.reshape(-1).astype(jnp.int32)
    offsets = jnp.arange(F, dtype=jnp.int32) * _CARD
    offs_pat = jnp.tile(offsets, _CB)
    out = _build_fm_call(B)(x_flat, offs_pat, emb_rm, linear_table)
    return out.reshape(B, 1) + bias[None, :]
